# baseline (speedup 1.0000x reference)
import dataclasses
import functools

import jax
import jax.numpy as jnp
from jax import lax
from jax.experimental import pallas as pl
from jax.experimental.pallas import tpu as pltpu
from jax.experimental.pallas import tpu_sc as plsc

_N = 10000
_D = 256
_E = 160000
_EP = 163840
_NP = 10240
_ZROW = 10016
_NC = 2
_NS = 16
_RANGE = _NP // _NC
_BR = 1280
_NB = _NP // _BR

_mesh = plsc.VectorSubcoreMesh(core_axis_name="c", subcore_axis_name="s")

_sc_params = pltpu.CompilerParams()
if "needs_layout_passes" in pltpu.CompilerParams.__dataclass_fields__:
    _sc_params = dataclasses.replace(_sc_params, needs_layout_passes=False)


def _deg_body(dst_hbm, degp_hbm, part, dbuf, stage, red, outbuf):
    c = lax.axis_index("c")
    s = lax.axis_index("s")
    zeros16 = jnp.zeros((16,), jnp.float32)
    ones16 = jnp.ones((16,), jnp.float32)

    @pl.loop(0, _NP, step=16)
    def _(i):
        part[pl.ds(i, 16)] = zeros16

    w = c * _NS + s
    per_w = _EP // (_NC * _NS)
    pltpu.sync_copy(dst_hbm.at[pl.ds(w * per_w, per_w)], dbuf)

    @pl.loop(0, per_w // 16)
    def _(i):
        idx16 = dbuf[pl.ds(i * 16, 16)]
        plsc.addupdate_scatter(part, [idx16], ones16)

    pltpu.sync_copy(part, stage.at[s])
    plsc.subcore_barrier()
    nps = _NP // _NS
    pltpu.sync_copy(stage.at[:, pl.ds(s * nps, nps)], red)

    @pl.loop(0, nps, step=16)
    def _(i):
        acc = red[0, pl.ds(i, 16)]
        for k in range(1, _NS):
            acc = acc + red[k, pl.ds(i, 16)]
        outbuf[pl.ds(i, 16)] = acc

    pltpu.sync_copy(outbuf, degp_hbm.at[pl.ds(c * _NP + s * nps, nps)])


_deg_call = pl.kernel(
    _deg_body,
    out_type=jax.ShapeDtypeStruct((_NC * _NP,), jnp.float32),
    mesh=_mesh,
    scratch_types=[
        pltpu.VMEM((_NP,), jnp.float32),
        pltpu.VMEM((_EP // (_NC * _NS),), jnp.int32),
        pltpu.VMEM_SHARED((_NS, _NP), jnp.float32),
        pltpu.VMEM((_NS, _NP // _NS), jnp.float32),
        pltpu.VMEM((_NP // _NS,), jnp.float32),
    ],
    compiler_params=_sc_params,
)


_K = 2
_CH = 128
_SCAN = 1024
_SUBR = 2560
_SHARE = _EP // _NS
_CAP = _SHARE + 3 * _CH


def _agg_body(h_hbm, src_hbm, dst_hbm, out_hbm, accl, accr, csrc, cdst,
              sscan, dscan, sidx, gbufs, gl, gr, nref, gsem, sseml, ssemr):
    c = lax.axis_index("c")
    s = lax.axis_index("s")
    rps = _SUBR // _NS

    def gather_start(q):
        idx = csrc.at[pl.ds(q * _CH, _CH)]
        pltpu.async_copy(h_hbm.at[idx], gbufs.at[0], gsem.at[0])

    def gather_wait(q):
        idx = csrc.at[pl.ds(q * _CH, _CH)]
        pltpu.make_async_copy(h_hbm.at[idx], gbufs.at[0], gsem.at[0]).wait()

    def split_stage(q):
        @pl.loop(0, _CH)
        def _(r):
            for j in range(8):
                gl[r, pl.ds(j * 16, 16)] = gbufs[0, r, pl.ds(j * 16, 16)]
                gr[r, pl.ds(j * 16, 16)] = gbufs[0, r,
                                                 pl.ds(128 + j * 16, 16)]
        for j in range(_CH // 16):
            sidx.at[0][pl.ds(j * 16, 16)] = cdst[pl.ds(q * _CH + j * 16, 16)]

    def scat_start():
        pltpu.async_copy(gl, accl.at[sidx.at[0]], sseml.at[0], add=True)
        pltpu.async_copy(gr, accr.at[sidx.at[0]], ssemr.at[0], add=True)

    def scat_wait():
        pltpu.make_async_copy(gl, accl.at[sidx.at[0]], sseml.at[0]).wait()
        pltpu.make_async_copy(gr, accr.at[sidx.at[0]], ssemr.at[0]).wait()

    zsrc = jnp.full((16,), _ZROW, jnp.int32)
    zdst = jnp.zeros((16,), jnp.int32)
    bb = gbufs.at[0].at[pl.ds(0, 32)]

    for u in range(2):
        lo = c * (_NP // _NC) + u * _SUBR

        for k in range(rps // 32):
            rows_l = pl.ds(s * rps + k * 32, 32)
            rows_g = pl.ds(lo + s * rps + k * 32, 32)
            pltpu.sync_copy(h_hbm.at[rows_g], bb)
            pltpu.sync_copy(bb.at[:, pl.ds(0, 128)], accl.at[rows_l])
            pltpu.sync_copy(bb.at[:, pl.ds(128, 128)], accr.at[rows_l])
        plsc.subcore_barrier()

        nref[0] = 0
        ebase = s * _SHARE

        @pl.loop(0, _SHARE // _SCAN)
        def _(ci):
            off = ebase + ci * _SCAN
            pltpu.sync_copy(src_hbm.at[pl.ds(off, _SCAN)], sscan)
            pltpu.sync_copy(dst_hbm.at[pl.ds(off, _SCAN)], dscan)

            @pl.loop(0, _SCAN // 16)
            def _(i):
                s16 = sscan[pl.ds(i * 16, 16)]
                d16 = dscan[pl.ds(i * 16, 16)]
                m = jnp.logical_and(d16 >= lo, d16 < lo + _SUBR)
                n = nref[0]
                plsc.store_compressed(csrc.at[pl.ds(n, 16)], s16, mask=m)
                plsc.store_compressed(cdst.at[pl.ds(n, 16)], d16 - lo,
                                      mask=m)
                nref[0] = n + jnp.max(plsc.all_reduce_population_count(m))

        n = nref[0]
        for j in range((3 * _CH) // 16):
            csrc[pl.ds(n + j * 16, 16)] = zsrc
            cdst[pl.ds(n + j * 16, 16)] = zdst

        rounds2 = jnp.maximum((n + _CH - 1) // _CH, 1)

        gather_start(0)
        gather_wait(0)
        split_stage(0)
        gather_start(1)
        scat_start()

        @pl.loop(1, rounds2)
        def _(q):
            gather_wait(q)
            scat_wait()
            split_stage(q)
            gather_start(q + 1)
            scat_start()

        scat_wait()
        gather_wait(rounds2)

        plsc.subcore_barrier()
        for k in range(rps // 32):
            rows_l = pl.ds(s * rps + k * 32, 32)
            rows_g = pl.ds(lo + s * rps + k * 32, 32)
            pltpu.sync_copy(accl.at[rows_l], bb.at[:, pl.ds(0, 128)])
            pltpu.sync_copy(accr.at[rows_l], bb.at[:, pl.ds(128, 128)])
            pltpu.sync_copy(bb, out_hbm.at[rows_g])
        plsc.subcore_barrier()


_agg_call = pl.kernel(
    _agg_body,
    out_type=jax.ShapeDtypeStruct((_NP, _D), jnp.float32),
    mesh=_mesh,
    scratch_types=[
        pltpu.VMEM_SHARED((_SUBR, 128), jnp.float32),
        pltpu.VMEM_SHARED((_SUBR, 128), jnp.float32),
        pltpu.VMEM((_CAP,), jnp.int32),
        pltpu.VMEM((_CAP,), jnp.int32),
        pltpu.VMEM((_SCAN,), jnp.int32),
        pltpu.VMEM((_SCAN,), jnp.int32),
        pltpu.VMEM((_K, _CH), jnp.int32),
        pltpu.VMEM((1, _CH, _D), jnp.float32),
        pltpu.VMEM((_CH, 128), jnp.float32),
        pltpu.VMEM((_CH, 128), jnp.float32),
        pltpu.SMEM((1,), jnp.int32),
        pltpu.SemaphoreType.DMA((1,)),
        pltpu.SemaphoreType.DMA((1,)),
        pltpu.SemaphoreType.DMA((1,)),
    ],
    compiler_params=_sc_params,
)


def _dis_body(degp_ref, out_ref):
    deg = degp_ref[0] + degp_ref[1] + 1.0
    row = lax.broadcasted_iota(jnp.int32, (_NP, 1), 0)
    out_ref[...] = jnp.where(row < _N, lax.rsqrt(deg)[:, None], 0.0)


def _dis(degp):
    return pl.pallas_call(
        _dis_body,
        grid=(1,),
        in_specs=[pl.BlockSpec((_NC, _NP), lambda r: (0, 0))],
        out_specs=pl.BlockSpec((_NP, 1), lambda r: (0, 0)),
        out_shape=jax.ShapeDtypeStruct((_NP, 1), jnp.float32),
    )(degp)


def _mm_scale_body(x_ref, w_ref, dis_ref, out_ref):
    h = jnp.dot(x_ref[...], w_ref[...], preferred_element_type=jnp.float32)
    out_ref[...] = h * dis_ref[...]


def _mm_scale(xp, w, disp):
    return pl.pallas_call(
        _mm_scale_body,
        grid=(_NB,),
        in_specs=[
            pl.BlockSpec((_BR, _D), lambda r: (r, 0)),
            pl.BlockSpec((_D, _D), lambda r: (0, 0)),
            pl.BlockSpec((_BR, 1), lambda r: (r, 0)),
        ],
        out_specs=pl.BlockSpec((_BR, _D), lambda r: (r, 0)),
        out_shape=jax.ShapeDtypeStruct((_NP, _D), jnp.float32),
    )(xp, w, disp)


def _bn_mm_body(agg_ref, dis_ref, b1_ref, g_ref, be_ref, w2_ref, out_ref,
                stats):
    p = pl.program_id(0)
    r = pl.program_id(1)
    y = agg_ref[...] * dis_ref[...] + b1_ref[...]

    @pl.when(jnp.logical_and(p == 0, r == 0))
    def _():
        stats[...] = jnp.zeros_like(stats)

    @pl.when(p == 0)
    def _():
        row = r * _BR + lax.broadcasted_iota(jnp.int32, (_BR, 1), 0)
        ym = jnp.where(row < _N, y, 0.0)
        stats[0, :] += jnp.sum(ym, axis=0)
        stats[1, :] += jnp.sum(ym * ym, axis=0)

    @pl.when(p == 1)
    def _():
        mean = stats[0, :] / _N
        var = stats[1, :] / _N - mean * mean
        inv = lax.rsqrt(var + 1e-5)
        yn = g_ref[...] * (y - mean) * inv + be_ref[...]
        h = jnp.maximum(yn, 0.0)
        h2 = jnp.dot(h, w2_ref[...], preferred_element_type=jnp.float32)
        out_ref[...] = h2 * dis_ref[...]


def _bn_mm(agg, disp, b1, g, be, w2):
    return pl.pallas_call(
        _bn_mm_body,
        grid=(2, _NB),
        in_specs=[
            pl.BlockSpec((_BR, _D), lambda p, r: (r, 0)),
            pl.BlockSpec((_BR, 1), lambda p, r: (r, 0)),
            pl.BlockSpec((_D,), lambda p, r: (0,)),
            pl.BlockSpec((_D,), lambda p, r: (0,)),
            pl.BlockSpec((_D,), lambda p, r: (0,)),
            pl.BlockSpec((_D, _D), lambda p, r: (0, 0)),
        ],
        out_specs=pl.BlockSpec((_BR, _D), lambda p, r: (r, 0)),
        out_shape=jax.ShapeDtypeStruct((_NP, _D), jnp.float32),
        scratch_shapes=[pltpu.VMEM((2, _D), jnp.float32)],
    )(agg, disp, b1, g, be, w2)


def _skip_body(x_ref, w_ref, b_ref, out_ref):
    out_ref[...] = (
        jnp.dot(x_ref[...], w_ref[...], preferred_element_type=jnp.float32)
        + b_ref[...]
    )


def _skip(xp, w, b):
    return pl.pallas_call(
        _skip_body,
        grid=(_NB,),
        in_specs=[
            pl.BlockSpec((_BR, _D), lambda r: (r, 0)),
            pl.BlockSpec((_D, _D), lambda r: (0, 0)),
            pl.BlockSpec((_D,), lambda r: (0,)),
        ],
        out_specs=pl.BlockSpec((_BR, _D), lambda r: (r, 0)),
        out_shape=jax.ShapeDtypeStruct((_NP, _D), jnp.float32),
    )(xp, w, b)


def _final_body(agg_ref, dis_ref, b2_ref, skip_ref, out_ref):
    y = agg_ref[...] * dis_ref[...]
    y = jnp.maximum(y + b2_ref[...], 0.0)
    out_ref[...] = jnp.maximum(y + skip_ref[...], 0.0)


def _final(agg, disp, b2, skip):
    return pl.pallas_call(
        _final_body,
        grid=(_N // 2000,),
        in_specs=[
            pl.BlockSpec((2000, _D), lambda r: (r, 0)),
            pl.BlockSpec((2000, 1), lambda r: (r, 0)),
            pl.BlockSpec((_D,), lambda r: (0,)),
            pl.BlockSpec((2000, _D), lambda r: (r, 0)),
        ],
        out_specs=pl.BlockSpec((2000, _D), lambda r: (r, 0)),
        out_shape=jax.ShapeDtypeStruct((_N, _D), jnp.float32),
    )(agg, disp, b2, skip)


def kernel(x, edge_index, W1, b1, W2, b2, bn_gamma, bn_beta, W_skip, b_skip):
    src = edge_index[0]
    dst = edge_index[1]
    pad = _EP - _E
    sflat = jnp.concatenate([src, jnp.full((pad,), _ZROW, jnp.int32)])
    dflat = jnp.concatenate([dst, jnp.full((pad,), _ZROW, jnp.int32)])
    xp = jnp.concatenate([x, jnp.zeros((_NP - _N, _D), jnp.float32)])

    degp = _deg_call(dflat).reshape(_NC, _NP)
    disp = _dis(degp)
    h1 = _mm_scale(xp, W1, disp)
    agg1 = _agg_call(h1, sflat, dflat)
    h2 = _bn_mm(agg1, disp, b1, bn_gamma, bn_beta, W2)
    agg2 = _agg_call(h2, sflat, dflat)
    skipp = _skip(xp, W_skip, b_skip)
    return _final(agg2, disp, b2, skipp)


# speedup vs baseline: 3.4502x; 3.4502x over previous
import dataclasses
import functools

import jax
import jax.numpy as jnp
from jax import lax
from jax.experimental import pallas as pl
from jax.experimental.pallas import tpu as pltpu
from jax.experimental.pallas import tpu_sc as plsc

_N = 10000
_D = 256
_E = 160000
_EP = 163840
_RR = _EP // 128
_NP = 10240
_TRASH = 10016
_NC = 2
_NS = 16
_BR = 1000
_NB = _N // _BR

_mesh = plsc.VectorSubcoreMesh(core_axis_name="c", subcore_axis_name="s")

_sc_params = pltpu.CompilerParams()
if "needs_layout_passes" in pltpu.CompilerParams.__dataclass_fields__:
    _sc_params = dataclasses.replace(_sc_params, needs_layout_passes=False)


def _deg_body(dst_hbm, degp_hbm, part, dbuf, stage, red, outbuf):
    c = lax.axis_index("c")
    s = lax.axis_index("s")
    zeros16 = jnp.zeros((16,), jnp.float32)
    ones16 = jnp.ones((16,), jnp.float32)

    @pl.loop(0, _NP, step=16)
    def _(i):
        part[pl.ds(i, 16)] = zeros16

    w = c * _NS + s
    per_w = _EP // (_NC * _NS)
    pltpu.sync_copy(dst_hbm.at[pl.ds(w * per_w, per_w)], dbuf)

    @pl.loop(0, per_w // 16)
    def _(i):
        idx16 = dbuf[pl.ds(i * 16, 16)]
        plsc.addupdate_scatter(part, [idx16], ones16)

    pltpu.sync_copy(part, stage.at[s])
    plsc.subcore_barrier()
    nps = _NP // _NS
    pltpu.sync_copy(stage.at[:, pl.ds(s * nps, nps)], red)

    @pl.loop(0, nps, step=16)
    def _(i):
        acc = red[0, pl.ds(i, 16)]
        for k in range(1, _NS):
            acc = acc + red[k, pl.ds(i, 16)]
        outbuf[pl.ds(i, 16)] = acc

    pltpu.sync_copy(outbuf, degp_hbm.at[pl.ds(c * _NP + s * nps, nps)])


_deg_call = pl.kernel(
    _deg_body,
    out_type=jax.ShapeDtypeStruct((_NC * _NP,), jnp.float32),
    mesh=_mesh,
    scratch_types=[
        pltpu.VMEM((_NP,), jnp.float32),
        pltpu.VMEM((_EP // (_NC * _NS),), jnp.int32),
        pltpu.VMEM_SHARED((_NS, _NP), jnp.float32),
        pltpu.VMEM((_NS, _NP // _NS), jnp.float32),
        pltpu.VMEM((_NP // _NS,), jnp.float32),
    ],
    compiler_params=_sc_params,
)


_K = 2
_IC = 40


def _agg_body(h_hbm, src_hbm, dst_hbm, out_hbm, acc, sbuf, dbuf, gbufs,
              gsem, ssem):
    c = lax.axis_index("c")
    s = lax.axis_index("s")
    nps = _NP // _NS
    rows_per_s = _RR // _NS

    for k in range(5):
        rows = pl.ds(s * nps + k * 128, 128)
        pltpu.sync_copy(h_hbm.at[c].at[rows], gbufs.at[0])
        pltpu.sync_copy(gbufs.at[0], acc.at[rows])
    plsc.subcore_barrier()

    def gather_start(r, k):
        pltpu.async_copy(h_hbm.at[c].at[sbuf.at[r]], gbufs.at[k], gsem.at[k])

    def gather_wait(r, k):
        pltpu.make_async_copy(
            h_hbm.at[c].at[sbuf.at[r]], gbufs.at[k], gsem.at[k]).wait()

    def scat_start(r, k):
        pltpu.async_copy(gbufs.at[k], acc.at[dbuf.at[r]], ssem.at[k],
                         add=True)

    def scat_wait(r, k):
        pltpu.make_async_copy(
            gbufs.at[k], acc.at[dbuf.at[r]], ssem.at[k]).wait()

    for ci in range(rows_per_s // _IC):
        rows = pl.ds(s * rows_per_s + ci * _IC, _IC)
        pltpu.sync_copy(src_hbm.at[rows], sbuf)
        pltpu.sync_copy(dst_hbm.at[rows], dbuf)

        for k in range(_K):
            gather_start(k, k)

        ng = _IC // _K

        @pl.loop(0, ng - 1)
        def _(g):
            base = g * _K
            for k in range(_K):
                gather_wait(base + k, k)
                scat_start(base + k, k)
            for k in range(_K):
                scat_wait(base + k, k)
                gather_start(base + _K + k, k)

        last = (ng - 1) * _K
        for k in range(_K):
            gather_wait(last + k, k)
            scat_start(last + k, k)
        for k in range(_K):
            scat_wait(last + k, k)

    plsc.subcore_barrier()
    for k in range(5):
        rows = pl.ds(s * nps + k * 128, 128)
        pltpu.sync_copy(acc.at[rows], gbufs.at[0])
        pltpu.sync_copy(gbufs.at[0], out_hbm.at[c].at[rows])


_agg_call = pl.kernel(
    _agg_body,
    out_type=jax.ShapeDtypeStruct((_NC, _NP, 128), jnp.float32),
    mesh=_mesh,
    scratch_types=[
        pltpu.VMEM_SHARED((_NP, 128), jnp.float32),
        pltpu.VMEM((_IC, 128), jnp.int32),
        pltpu.VMEM((_IC, 128), jnp.int32),
        pltpu.VMEM((_K, 128, 128), jnp.float32),
        pltpu.SemaphoreType.DMA((_K,)),
        pltpu.SemaphoreType.DMA((_K,)),
    ],
)


def _dis_body(degp_ref, out_ref):
    deg = degp_ref[0] + degp_ref[1] + 1.0
    out_ref[...] = lax.rsqrt(deg)[:_N, None]


def _dis(degp):
    return pl.pallas_call(
        _dis_body,
        grid=(1,),
        in_specs=[pl.BlockSpec((_NC, _NP), lambda r: (0, 0))],
        out_specs=pl.BlockSpec((_N, 1), lambda r: (0, 0)),
        out_shape=jax.ShapeDtypeStruct((_N, 1), jnp.float32),
    )(degp)


def _mm_scale_body(x_ref, w_ref, dis_ref, out_ref):
    h = jnp.dot(x_ref[...], w_ref[...], preferred_element_type=jnp.float32)
    h = h * dis_ref[...]
    out_ref[0] = h[:, :128]
    out_ref[1] = h[:, 128:]


def _mm_scale(x, w, dis):
    return pl.pallas_call(
        _mm_scale_body,
        grid=(_NB,),
        in_specs=[
            pl.BlockSpec((_BR, _D), lambda r: (r, 0)),
            pl.BlockSpec((_D, _D), lambda r: (0, 0)),
            pl.BlockSpec((_BR, 1), lambda r: (r, 0)),
        ],
        out_specs=pl.BlockSpec((_NC, _BR, 128), lambda r: (0, r, 0)),
        out_shape=jax.ShapeDtypeStruct((_NC, _NP, 128), jnp.float32),
    )(x, w, dis)


def _bn_mm_body(agg_ref, dis_ref, b1_ref, g_ref, be_ref, w2_ref, out_ref,
                stats):
    p = pl.program_id(0)
    r = pl.program_id(1)
    y = jnp.concatenate([agg_ref[0], agg_ref[1]], axis=1) * dis_ref[...]
    y = y + b1_ref[...]

    @pl.when(jnp.logical_and(p == 0, r == 0))
    def _():
        stats[...] = jnp.zeros_like(stats)

    @pl.when(p == 0)
    def _():
        stats[0, :] += jnp.sum(y, axis=0)
        stats[1, :] += jnp.sum(y * y, axis=0)

    @pl.when(p == 1)
    def _():
        mean = stats[0, :] / _N
        var = stats[1, :] / _N - mean * mean
        inv = lax.rsqrt(var + 1e-5)
        yn = g_ref[...] * (y - mean) * inv + be_ref[...]
        h = jnp.maximum(yn, 0.0)
        h2 = jnp.dot(h, w2_ref[...], preferred_element_type=jnp.float32)
        h2 = h2 * dis_ref[...]
        out_ref[0] = h2[:, :128]
        out_ref[1] = h2[:, 128:]


def _bn_mm(agg, dis, b1, g, be, w2):
    return pl.pallas_call(
        _bn_mm_body,
        grid=(2, _NB),
        in_specs=[
            pl.BlockSpec((_NC, _BR, 128), lambda p, r: (0, r, 0)),
            pl.BlockSpec((_BR, 1), lambda p, r: (r, 0)),
            pl.BlockSpec((_D,), lambda p, r: (0,)),
            pl.BlockSpec((_D,), lambda p, r: (0,)),
            pl.BlockSpec((_D,), lambda p, r: (0,)),
            pl.BlockSpec((_D, _D), lambda p, r: (0, 0)),
        ],
        out_specs=pl.BlockSpec((_NC, _BR, 128), lambda p, r: (0, r, 0)),
        out_shape=jax.ShapeDtypeStruct((_NC, _NP, 128), jnp.float32),
        scratch_shapes=[pltpu.VMEM((2, _D), jnp.float32)],
    )(agg, dis, b1, g, be, w2)


def _skip_body(x_ref, w_ref, b_ref, out_ref):
    out_ref[...] = (
        jnp.dot(x_ref[...], w_ref[...], preferred_element_type=jnp.float32)
        + b_ref[...]
    )


def _skip(x, w, b):
    return pl.pallas_call(
        _skip_body,
        grid=(_NB,),
        in_specs=[
            pl.BlockSpec((_BR, _D), lambda r: (r, 0)),
            pl.BlockSpec((_D, _D), lambda r: (0, 0)),
            pl.BlockSpec((_D,), lambda r: (0,)),
        ],
        out_specs=pl.BlockSpec((_BR, _D), lambda r: (r, 0)),
        out_shape=jax.ShapeDtypeStruct((_N, _D), jnp.float32),
    )(x, w, b)


def _final_body(agg_ref, dis_ref, b2_ref, skip_ref, out_ref):
    y = jnp.concatenate([agg_ref[0], agg_ref[1]], axis=1) * dis_ref[...]
    y = jnp.maximum(y + b2_ref[...], 0.0)
    out_ref[...] = jnp.maximum(y + skip_ref[...], 0.0)


def _final(agg, dis, b2, skip):
    return pl.pallas_call(
        _final_body,
        grid=(_NB,),
        in_specs=[
            pl.BlockSpec((_NC, _BR, 128), lambda r: (0, r, 0)),
            pl.BlockSpec((_BR, 1), lambda r: (r, 0)),
            pl.BlockSpec((_D,), lambda r: (0,)),
            pl.BlockSpec((_BR, _D), lambda r: (r, 0)),
        ],
        out_specs=pl.BlockSpec((_BR, _D), lambda r: (r, 0)),
        out_shape=jax.ShapeDtypeStruct((_N, _D), jnp.float32),
    )(agg, dis, b2, skip)


def kernel(x, edge_index, W1, b1, W2, b2, bn_gamma, bn_beta, W_skip, b_skip):
    src = edge_index[0]
    dst = edge_index[1]
    pad = _EP - _E
    src2d = jnp.concatenate(
        [src, jnp.zeros((pad,), jnp.int32)]).reshape(_RR, 128)
    dst2d = jnp.concatenate(
        [dst, jnp.full((pad,), _TRASH, jnp.int32)]).reshape(_RR, 128)

    degp = _deg_call(dst2d.reshape(_EP)).reshape(_NC, _NP)
    dis = _dis(degp)
    h1p = _mm_scale(x, W1, dis)
    agg1 = _agg_call(h1p, src2d, dst2d)
    h2p = _bn_mm(agg1, dis, b1, bn_gamma, bn_beta, W2)
    agg2 = _agg_call(h2p, src2d, dst2d)
    skip = _skip(x, W_skip, b_skip)
    return _final(agg2, dis, b2, skip)


# speedup vs baseline: 3.4562x; 1.0017x over previous
import dataclasses
import functools

import jax
import jax.numpy as jnp
from jax import lax
from jax.experimental import pallas as pl
from jax.experimental.pallas import tpu as pltpu
from jax.experimental.pallas import tpu_sc as plsc

_N = 10000
_D = 256
_E = 160000
_EP = 163840
_RR = _EP // 128
_NP = 10240
_TRASH = 10016
_NC = 2
_NS = 16
_BR = 1000
_NB = _N // _BR

_mesh = plsc.VectorSubcoreMesh(core_axis_name="c", subcore_axis_name="s")

_sc_params = pltpu.CompilerParams()
if "needs_layout_passes" in pltpu.CompilerParams.__dataclass_fields__:
    _sc_params = dataclasses.replace(_sc_params, needs_layout_passes=False)


def _deg_body(dst_hbm, degp_hbm, part, dbuf, stage, red, outbuf):
    c = lax.axis_index("c")
    s = lax.axis_index("s")
    zeros16 = jnp.zeros((16,), jnp.float32)
    ones16 = jnp.ones((16,), jnp.float32)

    @pl.loop(0, _NP, step=16)
    def _(i):
        part[pl.ds(i, 16)] = zeros16

    w = c * _NS + s
    per_w = _EP // (_NC * _NS)
    pltpu.sync_copy(dst_hbm.at[pl.ds(w * per_w, per_w)], dbuf)

    @pl.loop(0, per_w // 16)
    def _(i):
        idx16 = dbuf[pl.ds(i * 16, 16)]
        plsc.addupdate_scatter(part, [idx16], ones16)

    pltpu.sync_copy(part, stage.at[s])
    plsc.subcore_barrier()
    nps = _NP // _NS
    pltpu.sync_copy(stage.at[:, pl.ds(s * nps, nps)], red)

    @pl.loop(0, nps, step=16)
    def _(i):
        acc = red[0, pl.ds(i, 16)]
        for k in range(1, _NS):
            acc = acc + red[k, pl.ds(i, 16)]
        outbuf[pl.ds(i, 16)] = acc

    pltpu.sync_copy(outbuf, degp_hbm.at[pl.ds(c * _NP + s * nps, nps)])


_deg_call = pl.kernel(
    _deg_body,
    out_type=jax.ShapeDtypeStruct((_NC * _NP,), jnp.float32),
    mesh=_mesh,
    scratch_types=[
        pltpu.VMEM((_NP,), jnp.float32),
        pltpu.VMEM((_EP // (_NC * _NS),), jnp.int32),
        pltpu.VMEM_SHARED((_NS, _NP), jnp.float32),
        pltpu.VMEM((_NS, _NP // _NS), jnp.float32),
        pltpu.VMEM((_NP // _NS,), jnp.float32),
    ],
    compiler_params=_sc_params,
)


_K = 2
_IC = 40


def _agg_body(h_hbm, src_hbm, dst_hbm, out_hbm, acc, sbuf, dbuf, gbufs,
              gsem, ssem):
    c = lax.axis_index("c")
    s = lax.axis_index("s")
    nps = _NP // _NS
    rows_per_s = _RR // _NS

    for k in range(5):
        rows = pl.ds(s * nps + k * 128, 128)
        pltpu.sync_copy(h_hbm.at[c].at[rows], gbufs.at[0])
        pltpu.sync_copy(gbufs.at[0], acc.at[rows])
    plsc.subcore_barrier()

    def gather_start(r, k):
        pltpu.async_copy(h_hbm.at[c].at[sbuf.at[r]], gbufs.at[k], gsem.at[k])

    def gather_wait(r, k):
        pltpu.make_async_copy(
            h_hbm.at[c].at[sbuf.at[r]], gbufs.at[k], gsem.at[k]).wait()

    def scat_start(r, k):
        pltpu.async_copy(gbufs.at[k], acc.at[dbuf.at[r]], ssem.at[k],
                         add=True)

    def scat_wait(r, k):
        pltpu.make_async_copy(
            gbufs.at[k], acc.at[dbuf.at[r]], ssem.at[k]).wait()

    for ci in range(rows_per_s // _IC):
        rows = pl.ds(s * rows_per_s + ci * _IC, _IC)
        pltpu.sync_copy(src_hbm.at[rows], sbuf)
        pltpu.sync_copy(dst_hbm.at[rows], dbuf)

        for k in range(_K):
            gather_start(k, k)

        ng = _IC // _K

        @pl.loop(0, ng - 1)
        def _(g):
            base = g * _K
            for k in range(_K):
                gather_wait(base + k, k)
                scat_start(base + k, k)
            for k in range(_K):
                scat_wait(base + k, k)
                gather_start(base + _K + k, k)

        last = (ng - 1) * _K
        for k in range(_K):
            gather_wait(last + k, k)
            scat_start(last + k, k)
        for k in range(_K):
            scat_wait(last + k, k)

    plsc.subcore_barrier()
    for k in range(5):
        rows = pl.ds(s * nps + k * 128, 128)
        pltpu.sync_copy(acc.at[rows], gbufs.at[0])
        pltpu.sync_copy(gbufs.at[0], out_hbm.at[c].at[rows])


_agg_call = pl.kernel(
    _agg_body,
    out_type=jax.ShapeDtypeStruct((_NC, _NP, 128), jnp.float32),
    mesh=_mesh,
    scratch_types=[
        pltpu.VMEM_SHARED((_NP, 128), jnp.float32),
        pltpu.VMEM((_IC, 128), jnp.int32),
        pltpu.VMEM((_IC, 128), jnp.int32),
        pltpu.VMEM((_K, 128, 128), jnp.float32),
        pltpu.SemaphoreType.DMA((_K,)),
        pltpu.SemaphoreType.DMA((_K,)),
    ],
)


def _dis_body(degp_ref, out_ref):
    deg = degp_ref[0] + degp_ref[1] + 1.0
    out_ref[...] = lax.rsqrt(deg)[:_N, None]


def _dis(degp):
    return pl.pallas_call(
        _dis_body,
        grid=(1,),
        in_specs=[pl.BlockSpec((_NC, _NP), lambda r: (0, 0))],
        out_specs=pl.BlockSpec((_N, 1), lambda r: (0, 0)),
        out_shape=jax.ShapeDtypeStruct((_N, 1), jnp.float32),
    )(degp)


def _mm_scale_body(x_ref, w_ref, dis_ref, out_ref):
    h = jnp.dot(x_ref[...], w_ref[...], preferred_element_type=jnp.float32)
    h = h * dis_ref[...]
    out_ref[0] = h[:, :128]
    out_ref[1] = h[:, 128:]


def _mm_scale(x, w, dis):
    return pl.pallas_call(
        _mm_scale_body,
        grid=(_NB,),
        in_specs=[
            pl.BlockSpec((_BR, _D), lambda r: (r, 0)),
            pl.BlockSpec((_D, _D), lambda r: (0, 0)),
            pl.BlockSpec((_BR, 1), lambda r: (r, 0)),
        ],
        out_specs=pl.BlockSpec((_NC, _BR, 128), lambda r: (0, r, 0)),
        out_shape=jax.ShapeDtypeStruct((_NC, _NP, 128), jnp.float32),
    )(x, w, dis)


def _bn_mm_body(agg_ref, dis_ref, b1_ref, g_ref, be_ref, w2_ref, out_ref,
                stats):
    p = pl.program_id(0)
    r = pl.program_id(1)
    y = jnp.concatenate([agg_ref[0], agg_ref[1]], axis=1) * dis_ref[...]
    y = y + b1_ref[...]

    @pl.when(jnp.logical_and(p == 0, r == 0))
    def _():
        stats[...] = jnp.zeros_like(stats)

    @pl.when(p == 0)
    def _():
        stats[0, :] += jnp.sum(y, axis=0)
        stats[1, :] += jnp.sum(y * y, axis=0)

    @pl.when(p == 1)
    def _():
        mean = stats[0, :] / _N
        var = stats[1, :] / _N - mean * mean
        inv = lax.rsqrt(var + 1e-5)
        yn = g_ref[...] * (y - mean) * inv + be_ref[...]
        h = jnp.maximum(yn, 0.0)
        h2 = jnp.dot(h, w2_ref[...], preferred_element_type=jnp.float32)
        h2 = h2 * dis_ref[...]
        out_ref[0] = h2[:, :128]
        out_ref[1] = h2[:, 128:]


def _bn_mm(agg, dis, b1, g, be, w2):
    return pl.pallas_call(
        _bn_mm_body,
        grid=(2, _NB),
        in_specs=[
            pl.BlockSpec((_NC, _BR, 128), lambda p, r: (0, r, 0)),
            pl.BlockSpec((_BR, 1), lambda p, r: (r, 0)),
            pl.BlockSpec((_D,), lambda p, r: (0,)),
            pl.BlockSpec((_D,), lambda p, r: (0,)),
            pl.BlockSpec((_D,), lambda p, r: (0,)),
            pl.BlockSpec((_D, _D), lambda p, r: (0, 0)),
        ],
        out_specs=pl.BlockSpec((_NC, _BR, 128), lambda p, r: (0, r, 0)),
        out_shape=jax.ShapeDtypeStruct((_NC, _NP, 128), jnp.float32),
        scratch_shapes=[pltpu.VMEM((2, _D), jnp.float32)],
    )(agg, dis, b1, g, be, w2)


def _final_body(agg_ref, dis_ref, b2_ref, x_ref, wsk_ref, bsk_ref, out_ref):
    y = jnp.concatenate([agg_ref[0], agg_ref[1]], axis=1) * dis_ref[...]
    y = jnp.maximum(y + b2_ref[...], 0.0)
    skip = jnp.dot(x_ref[...], wsk_ref[...],
                   preferred_element_type=jnp.float32) + bsk_ref[...]
    out_ref[...] = jnp.maximum(y + skip, 0.0)


def _final(agg, dis, b2, x, wsk, bsk):
    return pl.pallas_call(
        _final_body,
        grid=(_NB,),
        in_specs=[
            pl.BlockSpec((_NC, _BR, 128), lambda r: (0, r, 0)),
            pl.BlockSpec((_BR, 1), lambda r: (r, 0)),
            pl.BlockSpec((_D,), lambda r: (0,)),
            pl.BlockSpec((_BR, _D), lambda r: (r, 0)),
            pl.BlockSpec((_D, _D), lambda r: (0, 0)),
            pl.BlockSpec((_D,), lambda r: (0,)),
        ],
        out_specs=pl.BlockSpec((_BR, _D), lambda r: (r, 0)),
        out_shape=jax.ShapeDtypeStruct((_N, _D), jnp.float32),
    )(agg, dis, b2, x, wsk, bsk)


def kernel(x, edge_index, W1, b1, W2, b2, bn_gamma, bn_beta, W_skip, b_skip):
    src = edge_index[0]
    dst = edge_index[1]
    pad = _EP - _E
    src2d = jnp.concatenate(
        [src, jnp.zeros((pad,), jnp.int32)]).reshape(_RR, 128)
    dst2d = jnp.concatenate(
        [dst, jnp.full((pad,), _TRASH, jnp.int32)]).reshape(_RR, 128)

    degp = _deg_call(dst2d.reshape(_EP)).reshape(_NC, _NP)
    dis = _dis(degp)
    h1p = _mm_scale(x, W1, dis)
    agg1 = _agg_call(h1p, src2d, dst2d)
    h2p = _bn_mm(agg1, dis, b1, bn_gamma, bn_beta, W2)
    agg2 = _agg_call(h2p, src2d, dst2d)
    return _final(agg2, dis, b2, x, W_skip, b_skip)
